# baseline (speedup 1.0000x reference)
import functools

import jax
import jax.numpy as jnp
from jax import lax
from jax.experimental import pallas as pl
from jax.experimental.pallas import tpu as pltpu
from jax.experimental.pallas import tpu_sc as plsc

_IPB = 4
_D = 128


def _sc_gather(n_idx):
    mesh = plsc.VectorSubcoreMesh(core_axis_name="c", subcore_axis_name="s")

    @functools.partial(
        pl.kernel, mesh=mesh,
        out_type=jax.ShapeDtypeStruct((n_idx, _D), jnp.float32),
        scratch_types=[
            pltpu.VMEM((n_idx,), jnp.int32),
            pltpu.VMEM((n_idx, _D), jnp.float32),
            pltpu.SemaphoreType.DMA,
        ],
    )
    def k(idx_hbm, table_hbm, out_hbm, idx_v, rows_v, sem):
        @pl.when((lax.axis_index("s") == 0) & (lax.axis_index("c") == 0))
        def _():
            pltpu.sync_copy(idx_hbm, idx_v)
            pltpu.async_copy(table_hbm.at[idx_v], rows_v, sem).wait()
            pltpu.sync_copy(rows_v, out_hbm)

    return k


def _cc_body(wb_ref, tex_ref, out_ref):
    i = pl.program_id(0)
    for k in range(_IPB):
        r = i * _IPB + k
        for c in range(3):
            w = wb_ref[r, c]
            b = wb_ref[r, 3 + c]
            out_ref[k, c] = tex_ref[k, c] * w + b


@jax.jit
def kernel(texture, cam, weight, bias):
    B, C, H, W = texture.shape
    dt = texture.dtype
    n_cam = weight.shape[0] + 1
    w_full = jnp.concatenate(
        [jnp.ones((1, C), dt), weight.reshape(-1, C)], axis=0)
    b_full = jnp.concatenate(
        [jnp.zeros((1, C), dt), bias.reshape(-1, C)], axis=0)
    table = jnp.concatenate(
        [w_full, b_full, jnp.zeros((n_cam, _D - 2 * C), dt)], axis=1)
    cam32 = cam.astype(jnp.int32)

    wb = _sc_gather(B)(cam32, table)

    return pl.pallas_call(
        _cc_body,
        grid=(B // _IPB,),
        in_specs=[
            pl.BlockSpec(memory_space=pltpu.SMEM),
            pl.BlockSpec((_IPB, C, H, W), lambda i: (i, 0, 0, 0)),
        ],
        out_specs=pl.BlockSpec((_IPB, C, H, W), lambda i: (i, 0, 0, 0)),
        out_shape=jax.ShapeDtypeStruct(texture.shape, dt),
        compiler_params=pltpu.CompilerParams(
            dimension_semantics=("parallel",)),
    )(wb, texture)


# speedup vs baseline: 1.2711x; 1.2711x over previous
import jax
import jax.numpy as jnp
from jax.experimental import pallas as pl
from jax.experimental.pallas import tpu as pltpu

_IPB = 4


def _cc_body(cam_ref, w_ref, b_ref, tex_ref, out_ref):
    i = pl.program_id(0)
    for k in range(_IPB):
        ci = cam_ref[i * _IPB + k]
        for c in range(3):
            w = w_ref[ci, c]
            b = b_ref[ci, c]
            out_ref[k, c] = tex_ref[k, c] * w + b


@jax.jit
def kernel(texture, cam, weight, bias):
    B, C, H, W = texture.shape
    dt = texture.dtype
    w_full = jnp.concatenate(
        [jnp.ones((1, C), dt), weight.reshape(-1, C)], axis=0)
    b_full = jnp.concatenate(
        [jnp.zeros((1, C), dt), bias.reshape(-1, C)], axis=0)
    cam32 = cam.astype(jnp.int32)
    return pl.pallas_call(
        _cc_body,
        grid=(B // _IPB,),
        in_specs=[
            pl.BlockSpec(memory_space=pltpu.SMEM),
            pl.BlockSpec(memory_space=pltpu.SMEM),
            pl.BlockSpec(memory_space=pltpu.SMEM),
            pl.BlockSpec((_IPB, C, H, W), lambda i: (i, 0, 0, 0)),
        ],
        out_specs=pl.BlockSpec((_IPB, C, H, W), lambda i: (i, 0, 0, 0)),
        out_shape=jax.ShapeDtypeStruct(texture.shape, dt),
        compiler_params=pltpu.CompilerParams(
            dimension_semantics=("parallel",)),
    )(cam32, w_full, b_full, texture)
